# baseline (speedup 1.0000x reference)
import jax
import jax.numpy as jnp
from jax.experimental import pallas as pl
from jax.experimental.pallas import tpu as pltpu

_UP = 10
_K = 5
_STD_FACTOR = 3.0
_QB = 256
_MB = 256


def _d2mat(q, p):
    qq = jnp.sum(q * q, axis=1, keepdims=True)
    pp = jnp.sum(p * p, axis=1)
    qp = jax.lax.dot_general(q, p, (((1,), (1,)), ((), ())),
                             preferred_element_type=jnp.float32)
    return qq + pp[None, :] - 2.0 * qp


def _min_onehot(cur, iota, n):
    m = jnp.min(cur, axis=1, keepdims=True)
    tie = cur == m
    j = jnp.min(jnp.where(tie, iota, n), axis=1, keepdims=True)
    return iota == j


def _topk_select(q, pts, d2, k):
    qb, n = d2.shape
    iota = jax.lax.broadcasted_iota(jnp.int32, (qb, n), 1)
    out = []
    cur = d2
    for _ in range(k):
        onehot = _min_onehot(cur, iota, n)
        psel = jax.lax.dot_general(onehot.astype(jnp.float32), pts,
                                   (((1,), (0,)), ((), ())),
                                   preferred_element_type=jnp.float32)
        diff = q - psel
        dk = jnp.sum(diff * diff, axis=1)
        out.append((dk, psel))
        cur = jnp.where(onehot, jnp.inf, cur)
    return out


def _scalar_mask(val, shape):
    r = jax.lax.broadcasted_iota(jnp.int32, shape, 0)
    c = jax.lax.broadcasted_iota(jnp.int32, shape, 1)
    return jnp.where((r == 0) & (c == 0), val, 0.0)


_IDX_BITS = 12
_IDX_MASK = -(1 << _IDX_BITS)
_KEY_MAX = 2147483647


def _packed_keys(q, qq, pts):
    pp = jnp.sum(pts * pts, axis=1)
    qp2 = jax.lax.dot_general(q, pts + pts, (((1,), (1,)), ((), ())),
                              preferred_element_type=jnp.float32)
    d2 = jnp.maximum(qq + pp[None, :] - qp2, 1e-35)
    iota = jax.lax.broadcasted_iota(jnp.int32, d2.shape, 1)
    packed = (jax.lax.bitcast_convert_type(d2, jnp.int32)
              & jnp.int32(_IDX_MASK)) | iota
    return jax.lax.bitcast_convert_type(packed, jnp.float32)


def _key_value(m):
    mi = jax.lax.bitcast_convert_type(m, jnp.int32) & jnp.int32(_IDX_MASK)
    return jax.lax.bitcast_convert_type(mi, jnp.float32)


def _top5_mins(keys):
    kmins = [jnp.min(keys, axis=1, keepdims=True)]
    for _ in range(_K - 1):
        flt = jnp.where(keys > kmins[-1], keys, jnp.float32(jnp.inf))
        kmins.append(jnp.min(flt, axis=1, keepdims=True))
    return kmins


def _weight_mat(keys, kmins, ws):
    acc = jnp.zeros(keys.shape, jnp.float32)
    for m, w in zip(kmins, ws):
        acc = jnp.where(keys == m, w, acc)
    return acc


def _geo_kernel(q_ref, tgt_ref, src_ref, out_ref):
    b = pl.program_id(0)
    qi = pl.program_id(1)

    @pl.when(jnp.logical_and(b == 0, qi == 0))
    def _():
        out_ref[...] = jnp.zeros_like(out_ref)

    q = q_ref[0]
    tp = tgt_ref[0]
    sp = src_ref[0]
    qq = jnp.sum(q * q, axis=1, keepdims=True)

    keys_t = _packed_keys(q, qq, tp)
    kmins_t = _top5_mins(keys_t)
    dks_t = [_key_value(m) for m in kmins_t]
    d1 = dks_t[0]
    w_list = [jnp.exp(d1 - dk) for dk in dks_t]
    udf_t = jnp.zeros_like(qq)
    for dk, w in zip(dks_t, w_list):
        udf_t = udf_t + jnp.sqrt(dk + 1e-10) * w
    wacc_t = _weight_mat(keys_t, kmins_t, w_list)

    z = w_list[0]
    for w in w_list[1:]:
        z = z + w
    inv_z = 1.0 / z

    keys_s = _packed_keys(q, qq, sp)
    kmins_s = _top5_mins(keys_s)
    udf_s = jnp.zeros_like(qq)
    for m, w in zip(kmins_s, w_list):
        udf_s = udf_s + jnp.sqrt(_key_value(m) + 1e-10) * w
    wacc_s = _weight_mat(keys_s, kmins_s, w_list)

    gdiff = (jax.lax.dot_general(wacc_t, tp, (((1,), (0,)), ((), ())),
                                 preferred_element_type=jnp.float32) -
             jax.lax.dot_general(wacc_s, sp, (((1,), (0,)), ((), ())),
                                 preferred_element_type=jnp.float32))
    err = (jnp.abs(udf_t - udf_s) +
           jnp.sum(jnp.abs(gdiff), axis=1, keepdims=True)) * inv_z
    s = jnp.sum(err)
    out_ref[...] += _scalar_mask(s, out_ref.shape)


def _std_kernel(tq_ref, tp_ref, out_ref):
    tq = tq_ref[0]
    tp = tp_ref[0]
    d2 = _d2mat(tq, tp)
    mb, n = d2.shape
    iota = jax.lax.broadcasted_iota(jnp.int32, (mb, n), 1)
    oh1 = _min_onehot(d2, iota, n)
    cur = jnp.where(oh1, jnp.inf, d2)
    oh2 = _min_onehot(cur, iota, n)
    psel = jax.lax.dot_general(oh2.astype(jnp.float32), tp,
                               (((1,), (0,)), ((), ())),
                               preferred_element_type=jnp.float32)
    diff = tq - psel
    dk = jnp.sum(diff * diff, axis=1)
    std = jnp.sqrt(dk + 1e-10) * _STD_FACTOR
    out_ref[0] = jnp.broadcast_to(std[None, :], out_ref.shape[1:])


def _smooth_kernel(img_ref, out_ref):
    img = img_ref[...]
    h, w = img.shape[2], img.shape[3]
    acc = jnp.zeros(img.shape[:2] + (h - 2, w - 2), jnp.float32)
    for i in range(3):
        for j in range(3):
            acc = acc + img[:, :, i:i + h - 2, j:j + w - 2]
    mean = acc / 9.0
    mid = img[:, :, 1:h - 1, 1:w - 1]
    val = jnp.mean(jnp.abs(mid - mean))
    out_ref[...] = _scalar_mask(val, out_ref.shape)


def kernel(src, tgt, grid, ep, maxep, H, W):
    B, N, C = src.shape
    M = tgt.shape[1]
    Q = M * _UP + N

    std_full = pl.pallas_call(
        _std_kernel,
        grid=(B, M // _MB),
        in_specs=[
            pl.BlockSpec((1, _MB, C), lambda b, m: (b, m, 0)),
            pl.BlockSpec((1, M, C), lambda b, m: (b, 0, 0)),
        ],
        out_specs=pl.BlockSpec((1, 8, _MB), lambda b, m: (b, 0, m)),
        out_shape=jax.ShapeDtypeStruct((B, 8, M), jnp.float32),
    )(tgt, tgt)
    std = std_full[:, 0, :]

    qs = []
    for b in range(B):
        kb = jax.random.fold_in(jax.random.key(42), b)
        noise = jax.random.normal(kb, (M, _UP, C), dtype=jnp.float32)
        noise = noise * std[b][:, None, None]
        qn = (tgt[b][:, None, :] + noise).reshape(-1, C)
        qs.append(jnp.concatenate([qn, src[b]], axis=0))
    query = jnp.stack(qs)

    geo_out = pl.pallas_call(
        _geo_kernel,
        grid=(B, Q // _QB),
        in_specs=[
            pl.BlockSpec((1, _QB, C), lambda b, q: (b, q, 0)),
            pl.BlockSpec((1, M, C), lambda b, q: (b, 0, 0)),
            pl.BlockSpec((1, N, C), lambda b, q: (b, 0, 0)),
        ],
        out_specs=pl.BlockSpec((8, 128), lambda b, q: (0, 0)),
        out_shape=jax.ShapeDtypeStruct((8, 128), jnp.float32),
    )(query, tgt, src)
    geo_total = geo_out[0, 0] / B / Q

    Hs = 64
    Ws = N // Hs
    src_img = jnp.transpose(src, (0, 2, 1)).reshape(B, C, Hs, Ws)
    smth_out = pl.pallas_call(
        _smooth_kernel,
        out_specs=pl.BlockSpec((8, 128), lambda: (0, 0)),
        out_shape=jax.ShapeDtypeStruct((8, 128), jnp.float32),
    )(src_img)
    smth = smth_out[0, 0]

    wsmth = (1.0 / maxep) ** 2 * (ep - maxep) ** 2
    hw_ratio = (H * W) / (Hs * Ws)
    return (geo_total + wsmth * smth) * hw_ratio


# speedup vs baseline: 1.1972x; 1.1972x over previous
import jax
import jax.numpy as jnp
from jax.experimental import pallas as pl
from jax.experimental.pallas import tpu as pltpu
from jax.experimental.pallas import tpu_sc as plsc

_UP = 10
_K = 5
_STD_FACTOR = 3.0
_QB = 256
_MB = 256
_NC = 2
_NS = 16
_NW = _NC * _NS
_LANES = 16

_IDX_BITS = 12
_IDX_MASK = -(1 << _IDX_BITS)
_IDX_LOW = (1 << _IDX_BITS) - 1


def _d2mat(q, p):
    qq = jnp.sum(q * q, axis=1, keepdims=True)
    pp = jnp.sum(p * p, axis=1)
    qp = jax.lax.dot_general(q, p, (((1,), (1,)), ((), ())),
                             preferred_element_type=jnp.float32)
    return qq + pp[None, :] - 2.0 * qp


def _min_onehot(cur, iota, n):
    m = jnp.min(cur, axis=1, keepdims=True)
    tie = cur == m
    j = jnp.min(jnp.where(tie, iota, n), axis=1, keepdims=True)
    return iota == j


def _scalar_mask(val, shape):
    r = jax.lax.broadcasted_iota(jnp.int32, shape, 0)
    c = jax.lax.broadcasted_iota(jnp.int32, shape, 1)
    return jnp.where((r == 0) & (c == 0), val, 0.0)


def _packed_keys(q, qq, pts):
    pp = jnp.sum(pts * pts, axis=1)
    qp2 = jax.lax.dot_general(q, pts + pts, (((1,), (1,)), ((), ())),
                              preferred_element_type=jnp.float32)
    d2 = jnp.maximum(qq + pp[None, :] - qp2, 1e-35)
    iota = jax.lax.broadcasted_iota(jnp.int32, d2.shape, 1)
    packed = (jax.lax.bitcast_convert_type(d2, jnp.int32)
              & jnp.int32(_IDX_MASK)) | iota
    return jax.lax.bitcast_convert_type(packed, jnp.float32)


def _key_value(m):
    mi = jax.lax.bitcast_convert_type(m, jnp.int32) & jnp.int32(_IDX_MASK)
    return jax.lax.bitcast_convert_type(mi, jnp.float32)


def _key_index(m):
    return jax.lax.bitcast_convert_type(m, jnp.int32) & jnp.int32(_IDX_LOW)


def _top5_mins(keys):
    kmins = [jnp.min(keys, axis=1, keepdims=True)]
    for _ in range(_K - 1):
        flt = jnp.where(keys > kmins[-1], keys, jnp.float32(jnp.inf))
        kmins.append(jnp.min(flt, axis=1, keepdims=True))
    return kmins


def _geo_kernel(q_ref, tgt_ref, src_ref, out_ref, jt_ref, js_ref, wn_ref):
    b = pl.program_id(0)
    qi = pl.program_id(1)

    @pl.when(jnp.logical_and(b == 0, qi == 0))
    def _():
        out_ref[...] = jnp.zeros_like(out_ref)

    q = q_ref[0]
    tp = tgt_ref[0]
    sp = src_ref[0]
    qq = jnp.sum(q * q, axis=1, keepdims=True)

    keys_t = _packed_keys(q, qq, tp)
    kmins_t = _top5_mins(keys_t)
    dks_t = [_key_value(m) for m in kmins_t]
    d1 = dks_t[0]
    w_list = [jnp.exp(d1 - dk) for dk in dks_t]
    udf_t = jnp.zeros_like(qq)
    for dk, w in zip(dks_t, w_list):
        udf_t = udf_t + jnp.sqrt(dk + 1e-10) * w

    z = w_list[0]
    for w in w_list[1:]:
        z = z + w
    inv_z = 1.0 / z

    keys_s = _packed_keys(q, qq, sp)
    kmins_s = _top5_mins(keys_s)
    udf_s = jnp.zeros_like(qq)
    for m, w in zip(kmins_s, w_list):
        udf_s = udf_s + jnp.sqrt(_key_value(m) + 1e-10) * w

    for k in range(_K):
        jt_ref[0, :, k:k + 1] = _key_index(kmins_t[k])
        js_ref[0, :, k:k + 1] = _key_index(kmins_s[k])
        wn_ref[0, :, k:k + 1] = w_list[k] * inv_z

    s = jnp.sum(jnp.abs(udf_t - udf_s) * inv_z)
    out_ref[...] += _scalar_mask(s, out_ref.shape)


def _grad_sc_kernel(jt_hbm, js_hbm, wn_hbm, tp_hbm, sp_hbm, out_hbm,
                    jt_v, js_v, wn_v, tp_v, sp_v, acc_v):
    cid = jax.lax.axis_index("c")
    sid = jax.lax.axis_index("s")
    wid = sid * _NC + cid
    nq = jt_v.shape[0] // 8
    b = wid // _NS
    qoff = (wid % _NS) * nq
    pltpu.sync_copy(jt_hbm.at[b, pl.ds(qoff * 8, nq * 8)], jt_v)
    pltpu.sync_copy(js_hbm.at[b, pl.ds(qoff * 8, nq * 8)], js_v)
    pltpu.sync_copy(wn_hbm.at[b, pl.ds(qoff * 8, nq * 8)], wn_v)
    pltpu.sync_copy(tp_hbm.at[b], tp_v)
    pltpu.sync_copy(sp_hbm.at[b], sp_v)

    lanes = jax.lax.broadcasted_iota(jnp.int32, (_LANES,), 0)

    def body(g, acc):
        qb = (lanes + g * _LANES) * 8
        a = [jnp.zeros((_LANES,), jnp.float32) for _ in range(3)]
        for k in range(_K):
            jtk4 = plsc.load_gather(jt_v, [qb + k]) * 4
            jsk4 = plsc.load_gather(js_v, [qb + k]) * 4
            wnk = plsc.load_gather(wn_v, [qb + k])
            for c in range(3):
                t = plsc.load_gather(tp_v, [jtk4 + c])
                s = plsc.load_gather(sp_v, [jsk4 + c])
                a[c] = a[c] + wnk * (t - s)
        return acc + jnp.abs(a[0]) + jnp.abs(a[1]) + jnp.abs(a[2])

    acc = jax.lax.fori_loop(0, (nq * 8) // 8 // _LANES, body,
                            jnp.zeros((_LANES,), jnp.float32))
    acc_v[...] = acc
    pltpu.sync_copy(acc_v, out_hbm.at[wid])


def _std_kernel(tq_ref, tp_ref, out_ref):
    tq = tq_ref[0]
    tp = tp_ref[0]
    d2 = _d2mat(tq, tp)
    mb, n = d2.shape
    iota = jax.lax.broadcasted_iota(jnp.int32, (mb, n), 1)
    oh1 = _min_onehot(d2, iota, n)
    cur = jnp.where(oh1, jnp.inf, d2)
    oh2 = _min_onehot(cur, iota, n)
    psel = jax.lax.dot_general(oh2.astype(jnp.float32), tp,
                               (((1,), (0,)), ((), ())),
                               preferred_element_type=jnp.float32)
    diff = tq - psel
    dk = jnp.sum(diff * diff, axis=1)
    std = jnp.sqrt(dk + 1e-10) * _STD_FACTOR
    out_ref[0] = jnp.broadcast_to(std[None, :], out_ref.shape[1:])


def _smooth_kernel(img_ref, out_ref):
    img = img_ref[...]
    h, w = img.shape[2], img.shape[3]
    acc = jnp.zeros(img.shape[:2] + (h - 2, w - 2), jnp.float32)
    for i in range(3):
        for j in range(3):
            acc = acc + img[:, :, i:i + h - 2, j:j + w - 2]
    mean = acc / 9.0
    mid = img[:, :, 1:h - 1, 1:w - 1]
    val = jnp.mean(jnp.abs(mid - mean))
    out_ref[...] = _scalar_mask(val, out_ref.shape)


def kernel(src, tgt, grid, ep, maxep, H, W):
    B, N, C = src.shape
    M = tgt.shape[1]
    Q = M * _UP + N

    std_full = pl.pallas_call(
        _std_kernel,
        grid=(B, M // _MB),
        in_specs=[
            pl.BlockSpec((1, _MB, C), lambda b, m: (b, m, 0)),
            pl.BlockSpec((1, M, C), lambda b, m: (b, 0, 0)),
        ],
        out_specs=pl.BlockSpec((1, 8, _MB), lambda b, m: (b, 0, m)),
        out_shape=jax.ShapeDtypeStruct((B, 8, M), jnp.float32),
    )(tgt, tgt)
    std = std_full[:, 0, :]

    qs = []
    for b in range(B):
        kb = jax.random.fold_in(jax.random.key(42), b)
        noise = jax.random.normal(kb, (M, _UP, C), dtype=jnp.float32)
        noise = noise * std[b][:, None, None]
        qn = (tgt[b][:, None, :] + noise).reshape(-1, C)
        qs.append(jnp.concatenate([qn, src[b]], axis=0))
    query = jnp.stack(qs)

    geo_out, jt, js, wn = pl.pallas_call(
        _geo_kernel,
        grid=(B, Q // _QB),
        in_specs=[
            pl.BlockSpec((1, _QB, C), lambda b, q: (b, q, 0)),
            pl.BlockSpec((1, M, C), lambda b, q: (b, 0, 0)),
            pl.BlockSpec((1, N, C), lambda b, q: (b, 0, 0)),
        ],
        out_specs=[
            pl.BlockSpec((8, 128), lambda b, q: (0, 0)),
            pl.BlockSpec((1, _QB, 8), lambda b, q: (b, q, 0)),
            pl.BlockSpec((1, _QB, 8), lambda b, q: (b, q, 0)),
            pl.BlockSpec((1, _QB, 8), lambda b, q: (b, q, 0)),
        ],
        out_shape=[
            jax.ShapeDtypeStruct((8, 128), jnp.float32),
            jax.ShapeDtypeStruct((B, Q, 8), jnp.int32),
            jax.ShapeDtypeStruct((B, Q, 8), jnp.int32),
            jax.ShapeDtypeStruct((B, Q, 8), jnp.float32),
        ],
    )(query, tgt, src)

    nq = B * Q // _NW
    tp_pad = jnp.concatenate(
        [tgt, jnp.zeros((B, M, 1), jnp.float32)], axis=2).reshape(B, M * 4)
    sp_pad = jnp.concatenate(
        [src, jnp.zeros((B, N, 1), jnp.float32)], axis=2).reshape(B, N * 4)
    grad_parts = pl.kernel(
        _grad_sc_kernel,
        out_type=jax.ShapeDtypeStruct((_NW, _LANES), jnp.float32),
        mesh=plsc.VectorSubcoreMesh(core_axis_name="c", subcore_axis_name="s"),
        compiler_params=pltpu.CompilerParams(needs_layout_passes=False),
        scratch_types=[
            pltpu.VMEM((nq * 8,), jnp.int32),
            pltpu.VMEM((nq * 8,), jnp.int32),
            pltpu.VMEM((nq * 8,), jnp.float32),
            pltpu.VMEM((M * 4,), jnp.float32),
            pltpu.VMEM((N * 4,), jnp.float32),
            pltpu.VMEM((_LANES,), jnp.float32),
        ],
    )(jt.reshape(B, Q * 8), js.reshape(B, Q * 8), wn.reshape(B, Q * 8),
      tp_pad, sp_pad)

    geo_total = (geo_out[0, 0] + jnp.sum(grad_parts)) / B / Q

    Hs = 64
    Ws = N // Hs
    src_img = jnp.transpose(src, (0, 2, 1)).reshape(B, C, Hs, Ws)
    smth_out = pl.pallas_call(
        _smooth_kernel,
        out_specs=pl.BlockSpec((8, 128), lambda: (0, 0)),
        out_shape=jax.ShapeDtypeStruct((8, 128), jnp.float32),
    )(src_img)
    smth = smth_out[0, 0]

    wsmth = (1.0 / maxep) ** 2 * (ep - maxep) ** 2
    hw_ratio = (H * W) / (Hs * Ws)
    return (geo_total + wsmth * smth) * hw_ratio


# speedup vs baseline: 1.2091x; 1.0100x over previous
import jax
import jax.numpy as jnp
from jax.experimental import pallas as pl
from jax.experimental.pallas import tpu as pltpu
from jax.experimental.pallas import tpu_sc as plsc

_UP = 10
_K = 5
_STD_FACTOR = 3.0
_QB = 256
_MB = 256
_NC = 2
_NS = 16
_NW = _NC * _NS
_LANES = 16

_IDX_BITS = 12
_IDX_MASK = -(1 << _IDX_BITS)
_IDX_LOW = (1 << _IDX_BITS) - 1


def _d2mat(q, p):
    qq = jnp.sum(q * q, axis=1, keepdims=True)
    pp = jnp.sum(p * p, axis=1)
    qp = jax.lax.dot_general(q, p, (((1,), (1,)), ((), ())),
                             preferred_element_type=jnp.float32)
    return qq + pp[None, :] - 2.0 * qp


def _min_onehot(cur, iota, n):
    m = jnp.min(cur, axis=1, keepdims=True)
    tie = cur == m
    j = jnp.min(jnp.where(tie, iota, n), axis=1, keepdims=True)
    return iota == j


def _scalar_mask(val, shape):
    r = jax.lax.broadcasted_iota(jnp.int32, shape, 0)
    c = jax.lax.broadcasted_iota(jnp.int32, shape, 1)
    return jnp.where((r == 0) & (c == 0), val, 0.0)


def _packed_keys(q, qq, pts):
    pp = jnp.sum(pts * pts, axis=1)
    qp2 = jax.lax.dot_general(q, pts + pts, (((1,), (1,)), ((), ())),
                              preferred_element_type=jnp.float32)
    d2 = jnp.maximum(qq + pp[None, :] - qp2, 1e-35)
    iota = jax.lax.broadcasted_iota(jnp.int32, d2.shape, 1)
    packed = (jax.lax.bitcast_convert_type(d2, jnp.int32)
              & jnp.int32(_IDX_MASK)) | iota
    return jax.lax.bitcast_convert_type(packed, jnp.float32)


def _key_value(m):
    mi = jax.lax.bitcast_convert_type(m, jnp.int32) & jnp.int32(_IDX_MASK)
    return jax.lax.bitcast_convert_type(mi, jnp.float32)


def _key_index(m):
    return jax.lax.bitcast_convert_type(m, jnp.int32) & jnp.int32(_IDX_LOW)


def _top5_mins(keys):
    kmins = [jnp.min(keys, axis=1, keepdims=True)]
    for _ in range(_K - 1):
        flt = jnp.where(keys > kmins[-1], keys, jnp.float32(jnp.inf))
        kmins.append(jnp.min(flt, axis=1, keepdims=True))
    return kmins


def _geo_kernel(q_ref, tgt_ref, src_ref, out_ref, jt_ref, js_ref, wn_ref):
    b = pl.program_id(0)
    qi = pl.program_id(1)

    @pl.when(jnp.logical_and(b == 0, qi == 0))
    def _():
        out_ref[...] = jnp.zeros_like(out_ref)

    q = q_ref[0]
    tp = tgt_ref[0]
    sp = src_ref[0]
    qq = jnp.sum(q * q, axis=1, keepdims=True)

    keys_t = _packed_keys(q, qq, tp)
    kmins_t = _top5_mins(keys_t)
    dks_t = [_key_value(m) for m in kmins_t]
    d1 = dks_t[0]
    w_list = [jnp.exp(d1 - dk) for dk in dks_t]
    udf_t = jnp.zeros_like(qq)
    for dk, w in zip(dks_t, w_list):
        udf_t = udf_t + jnp.sqrt(dk + 1e-10) * w

    z = w_list[0]
    for w in w_list[1:]:
        z = z + w
    inv_z = 1.0 / z

    keys_s = _packed_keys(q, qq, sp)
    kmins_s = _top5_mins(keys_s)
    udf_s = jnp.zeros_like(qq)
    for m, w in zip(kmins_s, w_list):
        udf_s = udf_s + jnp.sqrt(_key_value(m) + 1e-10) * w

    for k in range(_K):
        jt_ref[0, :, k:k + 1] = _key_index(kmins_t[k])
        js_ref[0, :, k:k + 1] = _key_index(kmins_s[k])
        wn_ref[0, :, k:k + 1] = w_list[k] * inv_z

    s = jnp.sum(jnp.abs(udf_t - udf_s) * inv_z)
    out_ref[...] += _scalar_mask(s, out_ref.shape)


def _grad_sc_kernel(jt_hbm, js_hbm, wn_hbm, tp_hbm, sp_hbm, out_hbm,
                    jt_v, js_v, wn_v, tp_v, sp_v, acc_v):
    cid = jax.lax.axis_index("c")
    sid = jax.lax.axis_index("s")
    wid = sid * _NC + cid
    nq = jt_v.shape[0] // 8
    b = wid // _NS
    qoff = (wid % _NS) * nq
    pltpu.sync_copy(jt_hbm.at[b, pl.ds(qoff * 8, nq * 8)], jt_v)
    pltpu.sync_copy(js_hbm.at[b, pl.ds(qoff * 8, nq * 8)], js_v)
    pltpu.sync_copy(wn_hbm.at[b, pl.ds(qoff * 8, nq * 8)], wn_v)
    pltpu.sync_copy(tp_hbm.at[b], tp_v)
    pltpu.sync_copy(sp_hbm.at[b], sp_v)

    lanes = jax.lax.broadcasted_iota(jnp.int32, (_LANES,), 0)

    def body(g, acc):
        qb = (lanes + g * _LANES) * 8
        a = [jnp.zeros((_LANES,), jnp.float32) for _ in range(3)]
        for k in range(_K):
            jtk4 = plsc.load_gather(jt_v, [qb + k]) * 4
            jsk4 = plsc.load_gather(js_v, [qb + k]) * 4
            wnk = plsc.load_gather(wn_v, [qb + k])
            for c in range(3):
                t = plsc.load_gather(tp_v, [jtk4 + c])
                s = plsc.load_gather(sp_v, [jsk4 + c])
                a[c] = a[c] + wnk * (t - s)
        return acc + jnp.abs(a[0]) + jnp.abs(a[1]) + jnp.abs(a[2])

    acc = jax.lax.fori_loop(0, (nq * 8) // 8 // _LANES, body,
                            jnp.zeros((_LANES,), jnp.float32))
    acc_v[...] = acc
    pltpu.sync_copy(acc_v, out_hbm.at[wid])


def _std_kernel(tq_ref, tp_ref, out_ref):
    tq = tq_ref[0]
    tp = tp_ref[0]
    d2 = _d2mat(tq, tp)
    mb, n = d2.shape
    iota = jax.lax.broadcasted_iota(jnp.int32, (mb, n), 1)
    oh1 = _min_onehot(d2, iota, n)
    cur = jnp.where(oh1, jnp.inf, d2)
    oh2 = _min_onehot(cur, iota, n)
    psel = jax.lax.dot_general(oh2.astype(jnp.float32), tp,
                               (((1,), (0,)), ((), ())),
                               preferred_element_type=jnp.float32)
    diff = tq - psel
    dk = jnp.sum(diff * diff, axis=1)
    std = jnp.sqrt(dk + 1e-10) * _STD_FACTOR
    out_ref[0] = jnp.broadcast_to(std[None, :], out_ref.shape[1:])


def _smooth_kernel(img_ref, out_ref):
    img = img_ref[...]
    h, w = img.shape[2], img.shape[3]
    acc = jnp.zeros(img.shape[:2] + (h - 2, w - 2), jnp.float32)
    for i in range(3):
        for j in range(3):
            acc = acc + img[:, :, i:i + h - 2, j:j + w - 2]
    mean = acc / 9.0
    mid = img[:, :, 1:h - 1, 1:w - 1]
    val = jnp.mean(jnp.abs(mid - mean))
    out_ref[...] = _scalar_mask(val, out_ref.shape)


def kernel(src, tgt, grid, ep, maxep, H, W):
    B, N, C = src.shape
    M = tgt.shape[1]
    Q = M * _UP + N

    std_full = pl.pallas_call(
        _std_kernel,
        grid=(B, M // _MB),
        in_specs=[
            pl.BlockSpec((1, _MB, C), lambda b, m: (b, m, 0)),
            pl.BlockSpec((1, M, C), lambda b, m: (b, 0, 0)),
        ],
        out_specs=pl.BlockSpec((1, 8, _MB), lambda b, m: (b, 0, m)),
        out_shape=jax.ShapeDtypeStruct((B, 8, M), jnp.float32),
    )(tgt, tgt)
    std = std_full[:, 0, :]

    qs = []
    for b in range(B):
        with jax.ensure_compile_time_eval():
            kb = jax.random.fold_in(jax.random.key(42), b)
            noise_raw = jax.random.normal(kb, (M, _UP, C), dtype=jnp.float32)
        noise = noise_raw * std[b][:, None, None]
        qn = (tgt[b][:, None, :] + noise).reshape(-1, C)
        qs.append(jnp.concatenate([qn, src[b]], axis=0))
    query = jnp.stack(qs)

    geo_out, jt, js, wn = pl.pallas_call(
        _geo_kernel,
        grid=(B, Q // _QB),
        in_specs=[
            pl.BlockSpec((1, _QB, C), lambda b, q: (b, q, 0)),
            pl.BlockSpec((1, M, C), lambda b, q: (b, 0, 0)),
            pl.BlockSpec((1, N, C), lambda b, q: (b, 0, 0)),
        ],
        out_specs=[
            pl.BlockSpec((8, 128), lambda b, q: (0, 0)),
            pl.BlockSpec((1, _QB, 8), lambda b, q: (b, q, 0)),
            pl.BlockSpec((1, _QB, 8), lambda b, q: (b, q, 0)),
            pl.BlockSpec((1, _QB, 8), lambda b, q: (b, q, 0)),
        ],
        out_shape=[
            jax.ShapeDtypeStruct((8, 128), jnp.float32),
            jax.ShapeDtypeStruct((B, Q, 8), jnp.int32),
            jax.ShapeDtypeStruct((B, Q, 8), jnp.int32),
            jax.ShapeDtypeStruct((B, Q, 8), jnp.float32),
        ],
    )(query, tgt, src)

    nq = B * Q // _NW
    tp_pad = jnp.concatenate(
        [tgt, jnp.zeros((B, M, 1), jnp.float32)], axis=2).reshape(B, M * 4)
    sp_pad = jnp.concatenate(
        [src, jnp.zeros((B, N, 1), jnp.float32)], axis=2).reshape(B, N * 4)
    grad_parts = pl.kernel(
        _grad_sc_kernel,
        out_type=jax.ShapeDtypeStruct((_NW, _LANES), jnp.float32),
        mesh=plsc.VectorSubcoreMesh(core_axis_name="c", subcore_axis_name="s"),
        compiler_params=pltpu.CompilerParams(needs_layout_passes=False),
        scratch_types=[
            pltpu.VMEM((nq * 8,), jnp.int32),
            pltpu.VMEM((nq * 8,), jnp.int32),
            pltpu.VMEM((nq * 8,), jnp.float32),
            pltpu.VMEM((M * 4,), jnp.float32),
            pltpu.VMEM((N * 4,), jnp.float32),
            pltpu.VMEM((_LANES,), jnp.float32),
        ],
    )(jt.reshape(B, Q * 8), js.reshape(B, Q * 8), wn.reshape(B, Q * 8),
      tp_pad, sp_pad)

    geo_total = (geo_out[0, 0] + jnp.sum(grad_parts)) / B / Q

    Hs = 64
    Ws = N // Hs
    src_img = jnp.transpose(src, (0, 2, 1)).reshape(B, C, Hs, Ws)
    smth_out = pl.pallas_call(
        _smooth_kernel,
        out_specs=pl.BlockSpec((8, 128), lambda: (0, 0)),
        out_shape=jax.ShapeDtypeStruct((8, 128), jnp.float32),
    )(src_img)
    smth = smth_out[0, 0]

    wsmth = (1.0 / maxep) ** 2 * (ep - maxep) ** 2
    hw_ratio = (H * W) / (Hs * Ws)
    return (geo_total + wsmth * smth) * hw_ratio


# speedup vs baseline: 1.3281x; 1.0984x over previous
import jax
import jax.numpy as jnp
from jax.experimental import pallas as pl
from jax.experimental.pallas import tpu as pltpu
from jax.experimental.pallas import tpu_sc as plsc

_UP = 10
_K = 5
_STD_FACTOR = 3.0
_QB = 512
_MB = 256
_NC = 2
_NS = 16
_NW = _NC * _NS
_LANES = 16

_IDX_BITS = 12
_IDX_MASK = -(1 << _IDX_BITS)
_IDX_LOW = (1 << _IDX_BITS) - 1


def _d2mat(q, p):
    qq = jnp.sum(q * q, axis=1, keepdims=True)
    pp = jnp.sum(p * p, axis=1)
    qp = jax.lax.dot_general(q, p, (((1,), (1,)), ((), ())),
                             preferred_element_type=jnp.float32)
    return qq + pp[None, :] - 2.0 * qp


def _min_onehot(cur, iota, n):
    m = jnp.min(cur, axis=1, keepdims=True)
    tie = cur == m
    j = jnp.min(jnp.where(tie, iota, n), axis=1, keepdims=True)
    return iota == j


def _scalar_mask(val, shape):
    r = jax.lax.broadcasted_iota(jnp.int32, shape, 0)
    c = jax.lax.broadcasted_iota(jnp.int32, shape, 1)
    return jnp.where((r == 0) & (c == 0), val, 0.0)


def _packed_keys(q, qq, pts):
    pp = jnp.sum(pts * pts, axis=1)
    qp2 = jax.lax.dot_general(q, pts + pts, (((1,), (1,)), ((), ())),
                              preferred_element_type=jnp.float32)
    d2 = jnp.maximum(qq + pp[None, :] - qp2, 1e-35)
    iota = jax.lax.broadcasted_iota(jnp.int32, d2.shape, 1)
    packed = (jax.lax.bitcast_convert_type(d2, jnp.int32)
              & jnp.int32(_IDX_MASK)) | iota
    return jax.lax.bitcast_convert_type(packed, jnp.float32)


def _key_value(m):
    mi = jax.lax.bitcast_convert_type(m, jnp.int32) & jnp.int32(_IDX_MASK)
    return jax.lax.bitcast_convert_type(mi, jnp.float32)


def _key_index(m):
    return jax.lax.bitcast_convert_type(m, jnp.int32) & jnp.int32(_IDX_LOW)


def _top5_mins(keys):
    kmins = [jnp.min(keys, axis=1, keepdims=True)]
    for _ in range(_K - 1):
        flt = jnp.where(keys > kmins[-1], keys, jnp.float32(jnp.inf))
        kmins.append(jnp.min(flt, axis=1, keepdims=True))
    return kmins


def _geo_kernel(q_ref, tgt_ref, src_ref, out_ref, jt_ref, js_ref, wn_ref):
    b = pl.program_id(0)
    qi = pl.program_id(1)

    @pl.when(jnp.logical_and(b == 0, qi == 0))
    def _():
        out_ref[...] = jnp.zeros_like(out_ref)

    q = q_ref[0]
    tp = tgt_ref[0]
    sp = src_ref[0]
    qq = jnp.sum(q * q, axis=1, keepdims=True)

    keys_t = _packed_keys(q, qq, tp)
    kmins_t = _top5_mins(keys_t)
    dks_t = [_key_value(m) for m in kmins_t]
    d1 = dks_t[0]
    w_list = [jnp.exp(d1 - dk) for dk in dks_t]
    udf_t = jnp.zeros_like(qq)
    for dk, w in zip(dks_t, w_list):
        udf_t = udf_t + jnp.sqrt(dk + 1e-10) * w

    z = w_list[0]
    for w in w_list[1:]:
        z = z + w
    inv_z = 1.0 / z

    keys_s = _packed_keys(q, qq, sp)
    kmins_s = _top5_mins(keys_s)
    udf_s = jnp.zeros_like(qq)
    for m, w in zip(kmins_s, w_list):
        udf_s = udf_s + jnp.sqrt(_key_value(m) + 1e-10) * w

    for k in range(_K):
        jt_ref[0, :, k:k + 1] = _key_index(kmins_t[k])
        js_ref[0, :, k:k + 1] = _key_index(kmins_s[k])
        wn_ref[0, :, k:k + 1] = w_list[k] * inv_z

    s = jnp.sum(jnp.abs(udf_t - udf_s) * inv_z)
    out_ref[...] += _scalar_mask(s, out_ref.shape)


def _grad_sc_kernel(jt_hbm, js_hbm, wn_hbm, tp_hbm, sp_hbm, out_hbm,
                    jt_v, js_v, wn_v, tp_v, sp_v, acc_v):
    cid = jax.lax.axis_index("c")
    sid = jax.lax.axis_index("s")
    wid = sid * _NC + cid
    nq = jt_v.shape[0] // 8
    b = wid // _NS
    qoff = (wid % _NS) * nq
    pltpu.sync_copy(jt_hbm.at[b, pl.ds(qoff * 8, nq * 8)], jt_v)
    pltpu.sync_copy(js_hbm.at[b, pl.ds(qoff * 8, nq * 8)], js_v)
    pltpu.sync_copy(wn_hbm.at[b, pl.ds(qoff * 8, nq * 8)], wn_v)
    pltpu.sync_copy(tp_hbm.at[b], tp_v)
    pltpu.sync_copy(sp_hbm.at[b], sp_v)

    lanes = jax.lax.broadcasted_iota(jnp.int32, (_LANES,), 0)

    def body(g, acc):
        qb = (lanes + g * _LANES) * 8
        a = [jnp.zeros((_LANES,), jnp.float32) for _ in range(3)]
        for k in range(_K):
            jtk4 = plsc.load_gather(jt_v, [qb + k]) * 4
            jsk4 = plsc.load_gather(js_v, [qb + k]) * 4
            wnk = plsc.load_gather(wn_v, [qb + k])
            for c in range(3):
                t = plsc.load_gather(tp_v, [jtk4 + c])
                s = plsc.load_gather(sp_v, [jsk4 + c])
                a[c] = a[c] + wnk * (t - s)
        return acc + jnp.abs(a[0]) + jnp.abs(a[1]) + jnp.abs(a[2])

    acc = jax.lax.fori_loop(0, (nq * 8) // 8 // _LANES, body,
                            jnp.zeros((_LANES,), jnp.float32))
    acc_v[...] = acc
    pltpu.sync_copy(acc_v, out_hbm.at[wid])


def _std_kernel(tq_ref, tp_ref, out_ref):
    tq = tq_ref[0]
    tp = tp_ref[0]
    d2 = _d2mat(tq, tp)
    mb, n = d2.shape
    iota = jax.lax.broadcasted_iota(jnp.int32, (mb, n), 1)
    oh1 = _min_onehot(d2, iota, n)
    cur = jnp.where(oh1, jnp.inf, d2)
    oh2 = _min_onehot(cur, iota, n)
    psel = jax.lax.dot_general(oh2.astype(jnp.float32), tp,
                               (((1,), (0,)), ((), ())),
                               preferred_element_type=jnp.float32)
    diff = tq - psel
    dk = jnp.sum(diff * diff, axis=1)
    std = jnp.sqrt(dk + 1e-10) * _STD_FACTOR
    out_ref[0] = jnp.broadcast_to(std[None, :], out_ref.shape[1:])


def _smooth_kernel(img_ref, out_ref):
    img = img_ref[...]
    h, w = img.shape[2], img.shape[3]
    acc = jnp.zeros(img.shape[:2] + (h - 2, w - 2), jnp.float32)
    for i in range(3):
        for j in range(3):
            acc = acc + img[:, :, i:i + h - 2, j:j + w - 2]
    mean = acc / 9.0
    mid = img[:, :, 1:h - 1, 1:w - 1]
    val = jnp.mean(jnp.abs(mid - mean))
    out_ref[...] = _scalar_mask(val, out_ref.shape)


def kernel(src, tgt, grid, ep, maxep, H, W):
    B, N, C = src.shape
    M = tgt.shape[1]
    Q = M * _UP + N

    std_full = pl.pallas_call(
        _std_kernel,
        grid=(B, M // _MB),
        in_specs=[
            pl.BlockSpec((1, _MB, C), lambda b, m: (b, m, 0)),
            pl.BlockSpec((1, M, C), lambda b, m: (b, 0, 0)),
        ],
        out_specs=pl.BlockSpec((1, 8, _MB), lambda b, m: (b, 0, m)),
        out_shape=jax.ShapeDtypeStruct((B, 8, M), jnp.float32),
    )(tgt, tgt)
    std = std_full[:, 0, :]

    qs = []
    for b in range(B):
        with jax.ensure_compile_time_eval():
            kb = jax.random.fold_in(jax.random.key(42), b)
            noise_raw = jax.random.normal(kb, (M, _UP, C), dtype=jnp.float32)
        noise = noise_raw * std[b][:, None, None]
        qn = (tgt[b][:, None, :] + noise).reshape(-1, C)
        qs.append(jnp.concatenate([qn, src[b]], axis=0))
    query = jnp.stack(qs)

    geo_out, jt, js, wn = pl.pallas_call(
        _geo_kernel,
        grid=(B, Q // _QB),
        in_specs=[
            pl.BlockSpec((1, _QB, C), lambda b, q: (b, q, 0)),
            pl.BlockSpec((1, M, C), lambda b, q: (b, 0, 0)),
            pl.BlockSpec((1, N, C), lambda b, q: (b, 0, 0)),
        ],
        out_specs=[
            pl.BlockSpec((8, 128), lambda b, q: (0, 0)),
            pl.BlockSpec((1, _QB, 8), lambda b, q: (b, q, 0)),
            pl.BlockSpec((1, _QB, 8), lambda b, q: (b, q, 0)),
            pl.BlockSpec((1, _QB, 8), lambda b, q: (b, q, 0)),
        ],
        out_shape=[
            jax.ShapeDtypeStruct((8, 128), jnp.float32),
            jax.ShapeDtypeStruct((B, Q, 8), jnp.int32),
            jax.ShapeDtypeStruct((B, Q, 8), jnp.int32),
            jax.ShapeDtypeStruct((B, Q, 8), jnp.float32),
        ],
    )(query, tgt, src)

    nq = B * Q // _NW
    tp_pad = jnp.concatenate(
        [tgt, jnp.zeros((B, M, 1), jnp.float32)], axis=2).reshape(B, M * 4)
    sp_pad = jnp.concatenate(
        [src, jnp.zeros((B, N, 1), jnp.float32)], axis=2).reshape(B, N * 4)
    grad_parts = pl.kernel(
        _grad_sc_kernel,
        out_type=jax.ShapeDtypeStruct((_NW, _LANES), jnp.float32),
        mesh=plsc.VectorSubcoreMesh(core_axis_name="c", subcore_axis_name="s"),
        compiler_params=pltpu.CompilerParams(needs_layout_passes=False),
        scratch_types=[
            pltpu.VMEM((nq * 8,), jnp.int32),
            pltpu.VMEM((nq * 8,), jnp.int32),
            pltpu.VMEM((nq * 8,), jnp.float32),
            pltpu.VMEM((M * 4,), jnp.float32),
            pltpu.VMEM((N * 4,), jnp.float32),
            pltpu.VMEM((_LANES,), jnp.float32),
        ],
    )(jt.reshape(B, Q * 8), js.reshape(B, Q * 8), wn.reshape(B, Q * 8),
      tp_pad, sp_pad)

    geo_total = (geo_out[0, 0] + jnp.sum(grad_parts)) / B / Q

    Hs = 64
    Ws = N // Hs
    src_img = jnp.transpose(src, (0, 2, 1)).reshape(B, C, Hs, Ws)
    smth_out = pl.pallas_call(
        _smooth_kernel,
        out_specs=pl.BlockSpec((8, 128), lambda: (0, 0)),
        out_shape=jax.ShapeDtypeStruct((8, 128), jnp.float32),
    )(src_img)
    smth = smth_out[0, 0]

    wsmth = (1.0 / maxep) ** 2 * (ep - maxep) ** 2
    hw_ratio = (H * W) / (Hs * Ws)
    return (geo_total + wsmth * smth) * hw_ratio


# speedup vs baseline: 1.3929x; 1.0488x over previous
import jax
import jax.numpy as jnp
from jax.experimental import pallas as pl
from jax.experimental.pallas import tpu as pltpu
from jax.experimental.pallas import tpu_sc as plsc

_UP = 10
_K = 5
_STD_FACTOR = 3.0
_QB = 1024
_MB = 256
_NC = 2
_NS = 16
_NW = _NC * _NS
_LANES = 16

_IDX_BITS = 12
_IDX_MASK = -(1 << _IDX_BITS)
_IDX_LOW = (1 << _IDX_BITS) - 1


def _d2mat(q, p):
    qq = jnp.sum(q * q, axis=1, keepdims=True)
    pp = jnp.sum(p * p, axis=1)
    qp = jax.lax.dot_general(q, p, (((1,), (1,)), ((), ())),
                             preferred_element_type=jnp.float32)
    return qq + pp[None, :] - 2.0 * qp


def _min_onehot(cur, iota, n):
    m = jnp.min(cur, axis=1, keepdims=True)
    tie = cur == m
    j = jnp.min(jnp.where(tie, iota, n), axis=1, keepdims=True)
    return iota == j


def _scalar_mask(val, shape):
    r = jax.lax.broadcasted_iota(jnp.int32, shape, 0)
    c = jax.lax.broadcasted_iota(jnp.int32, shape, 1)
    return jnp.where((r == 0) & (c == 0), val, 0.0)


def _packed_keys(q, qq, pts):
    pp = jnp.sum(pts * pts, axis=1)
    qp2 = jax.lax.dot_general(q, pts + pts, (((1,), (1,)), ((), ())),
                              preferred_element_type=jnp.float32)
    d2 = jnp.maximum(qq + pp[None, :] - qp2, 1e-35)
    iota = jax.lax.broadcasted_iota(jnp.int32, d2.shape, 1)
    packed = (jax.lax.bitcast_convert_type(d2, jnp.int32)
              & jnp.int32(_IDX_MASK)) | iota
    return jax.lax.bitcast_convert_type(packed, jnp.float32)


def _key_value(m):
    mi = jax.lax.bitcast_convert_type(m, jnp.int32) & jnp.int32(_IDX_MASK)
    return jax.lax.bitcast_convert_type(mi, jnp.float32)


def _key_index(m):
    return jax.lax.bitcast_convert_type(m, jnp.int32) & jnp.int32(_IDX_LOW)


def _top5_mins(keys):
    kmins = [jnp.min(keys, axis=1, keepdims=True)]
    for _ in range(_K - 1):
        flt = jnp.where(keys > kmins[-1], keys, jnp.float32(jnp.inf))
        kmins.append(jnp.min(flt, axis=1, keepdims=True))
    return kmins


def _geo_kernel(q_ref, tgt_ref, src_ref, out_ref, jt_ref, js_ref, wn_ref):
    b = pl.program_id(0)
    qi = pl.program_id(1)

    @pl.when(jnp.logical_and(b == 0, qi == 0))
    def _():
        out_ref[...] = jnp.zeros_like(out_ref)

    q = q_ref[0]
    tp = tgt_ref[0]
    sp = src_ref[0]
    qq = jnp.sum(q * q, axis=1, keepdims=True)

    keys_t = _packed_keys(q, qq, tp)
    kmins_t = _top5_mins(keys_t)
    dks_t = [_key_value(m) for m in kmins_t]
    d1 = dks_t[0]
    w_list = [jnp.exp(d1 - dk) for dk in dks_t]
    udf_t = jnp.zeros_like(qq)
    for dk, w in zip(dks_t, w_list):
        udf_t = udf_t + jnp.sqrt(dk + 1e-10) * w

    z = w_list[0]
    for w in w_list[1:]:
        z = z + w
    inv_z = 1.0 / z

    keys_s = _packed_keys(q, qq, sp)
    kmins_s = _top5_mins(keys_s)
    udf_s = jnp.zeros_like(qq)
    for m, w in zip(kmins_s, w_list):
        udf_s = udf_s + jnp.sqrt(_key_value(m) + 1e-10) * w

    for k in range(_K):
        jt_ref[0, :, k:k + 1] = _key_index(kmins_t[k])
        js_ref[0, :, k:k + 1] = _key_index(kmins_s[k])
        wn_ref[0, :, k:k + 1] = w_list[k] * inv_z

    s = jnp.sum(jnp.abs(udf_t - udf_s) * inv_z)
    out_ref[...] += _scalar_mask(s, out_ref.shape)


def _grad_sc_kernel(jt_hbm, js_hbm, wn_hbm, tp_hbm, sp_hbm, out_hbm,
                    jt_v, js_v, wn_v, tp_v, sp_v, acc_v):
    cid = jax.lax.axis_index("c")
    sid = jax.lax.axis_index("s")
    wid = sid * _NC + cid
    nq = jt_v.shape[0] // 8
    b = wid // _NS
    qoff = (wid % _NS) * nq
    pltpu.sync_copy(jt_hbm.at[b, pl.ds(qoff * 8, nq * 8)], jt_v)
    pltpu.sync_copy(js_hbm.at[b, pl.ds(qoff * 8, nq * 8)], js_v)
    pltpu.sync_copy(wn_hbm.at[b, pl.ds(qoff * 8, nq * 8)], wn_v)
    pltpu.sync_copy(tp_hbm.at[b], tp_v)
    pltpu.sync_copy(sp_hbm.at[b], sp_v)

    lanes = jax.lax.broadcasted_iota(jnp.int32, (_LANES,), 0)

    def body(g, acc):
        qb = (lanes + g * _LANES) * 8
        a = [jnp.zeros((_LANES,), jnp.float32) for _ in range(3)]
        for k in range(_K):
            jtk4 = plsc.load_gather(jt_v, [qb + k]) * 4
            jsk4 = plsc.load_gather(js_v, [qb + k]) * 4
            wnk = plsc.load_gather(wn_v, [qb + k])
            for c in range(3):
                t = plsc.load_gather(tp_v, [jtk4 + c])
                s = plsc.load_gather(sp_v, [jsk4 + c])
                a[c] = a[c] + wnk * (t - s)
        return acc + jnp.abs(a[0]) + jnp.abs(a[1]) + jnp.abs(a[2])

    acc = jax.lax.fori_loop(0, (nq * 8) // 8 // _LANES, body,
                            jnp.zeros((_LANES,), jnp.float32))
    acc_v[...] = acc
    pltpu.sync_copy(acc_v, out_hbm.at[wid])


def _std_kernel(tq_ref, tp_ref, out_ref):
    tq = tq_ref[0]
    tp = tp_ref[0]
    d2 = _d2mat(tq, tp)
    mb, n = d2.shape
    iota = jax.lax.broadcasted_iota(jnp.int32, (mb, n), 1)
    oh1 = _min_onehot(d2, iota, n)
    cur = jnp.where(oh1, jnp.inf, d2)
    oh2 = _min_onehot(cur, iota, n)
    psel = jax.lax.dot_general(oh2.astype(jnp.float32), tp,
                               (((1,), (0,)), ((), ())),
                               preferred_element_type=jnp.float32)
    diff = tq - psel
    dk = jnp.sum(diff * diff, axis=1)
    std = jnp.sqrt(dk + 1e-10) * _STD_FACTOR
    out_ref[0] = jnp.broadcast_to(std[None, :], out_ref.shape[1:])


def _smooth_kernel(img_ref, out_ref):
    img = img_ref[...]
    h, w = img.shape[2], img.shape[3]
    acc = jnp.zeros(img.shape[:2] + (h - 2, w - 2), jnp.float32)
    for i in range(3):
        for j in range(3):
            acc = acc + img[:, :, i:i + h - 2, j:j + w - 2]
    mean = acc / 9.0
    mid = img[:, :, 1:h - 1, 1:w - 1]
    val = jnp.mean(jnp.abs(mid - mean))
    out_ref[...] = _scalar_mask(val, out_ref.shape)


def kernel(src, tgt, grid, ep, maxep, H, W):
    B, N, C = src.shape
    M = tgt.shape[1]
    Q = M * _UP + N

    std_full = pl.pallas_call(
        _std_kernel,
        grid=(B, M // _MB),
        in_specs=[
            pl.BlockSpec((1, _MB, C), lambda b, m: (b, m, 0)),
            pl.BlockSpec((1, M, C), lambda b, m: (b, 0, 0)),
        ],
        out_specs=pl.BlockSpec((1, 8, _MB), lambda b, m: (b, 0, m)),
        out_shape=jax.ShapeDtypeStruct((B, 8, M), jnp.float32),
    )(tgt, tgt)
    std = std_full[:, 0, :]

    qs = []
    for b in range(B):
        with jax.ensure_compile_time_eval():
            kb = jax.random.fold_in(jax.random.key(42), b)
            noise_raw = jax.random.normal(kb, (M, _UP, C), dtype=jnp.float32)
        noise = noise_raw * std[b][:, None, None]
        qn = (tgt[b][:, None, :] + noise).reshape(-1, C)
        qs.append(jnp.concatenate([qn, src[b]], axis=0))
    query = jnp.stack(qs)

    geo_out, jt, js, wn = pl.pallas_call(
        _geo_kernel,
        grid=(B, Q // _QB),
        in_specs=[
            pl.BlockSpec((1, _QB, C), lambda b, q: (b, q, 0)),
            pl.BlockSpec((1, M, C), lambda b, q: (b, 0, 0)),
            pl.BlockSpec((1, N, C), lambda b, q: (b, 0, 0)),
        ],
        out_specs=[
            pl.BlockSpec((8, 128), lambda b, q: (0, 0)),
            pl.BlockSpec((1, _QB, 8), lambda b, q: (b, q, 0)),
            pl.BlockSpec((1, _QB, 8), lambda b, q: (b, q, 0)),
            pl.BlockSpec((1, _QB, 8), lambda b, q: (b, q, 0)),
        ],
        out_shape=[
            jax.ShapeDtypeStruct((8, 128), jnp.float32),
            jax.ShapeDtypeStruct((B, Q, 8), jnp.int32),
            jax.ShapeDtypeStruct((B, Q, 8), jnp.int32),
            jax.ShapeDtypeStruct((B, Q, 8), jnp.float32),
        ],
    )(query, tgt, src)

    nq = B * Q // _NW
    tp_pad = jnp.concatenate(
        [tgt, jnp.zeros((B, M, 1), jnp.float32)], axis=2).reshape(B, M * 4)
    sp_pad = jnp.concatenate(
        [src, jnp.zeros((B, N, 1), jnp.float32)], axis=2).reshape(B, N * 4)
    grad_parts = pl.kernel(
        _grad_sc_kernel,
        out_type=jax.ShapeDtypeStruct((_NW, _LANES), jnp.float32),
        mesh=plsc.VectorSubcoreMesh(core_axis_name="c", subcore_axis_name="s"),
        compiler_params=pltpu.CompilerParams(needs_layout_passes=False),
        scratch_types=[
            pltpu.VMEM((nq * 8,), jnp.int32),
            pltpu.VMEM((nq * 8,), jnp.int32),
            pltpu.VMEM((nq * 8,), jnp.float32),
            pltpu.VMEM((M * 4,), jnp.float32),
            pltpu.VMEM((N * 4,), jnp.float32),
            pltpu.VMEM((_LANES,), jnp.float32),
        ],
    )(jt.reshape(B, Q * 8), js.reshape(B, Q * 8), wn.reshape(B, Q * 8),
      tp_pad, sp_pad)

    geo_total = (geo_out[0, 0] + jnp.sum(grad_parts)) / B / Q

    Hs = 64
    Ws = N // Hs
    src_img = jnp.transpose(src, (0, 2, 1)).reshape(B, C, Hs, Ws)
    smth_out = pl.pallas_call(
        _smooth_kernel,
        out_specs=pl.BlockSpec((8, 128), lambda: (0, 0)),
        out_shape=jax.ShapeDtypeStruct((8, 128), jnp.float32),
    )(src_img)
    smth = smth_out[0, 0]

    wsmth = (1.0 / maxep) ** 2 * (ep - maxep) ** 2
    hw_ratio = (H * W) / (Hs * Ws)
    return (geo_total + wsmth * smth) * hw_ratio
